# baseline (speedup 1.0000x reference)
import jax
import jax.numpy as jnp
from jax.experimental import pallas as pl
from jax.experimental.pallas import tpu as pltpu

N = 10000
D = 128
CORES = 2
HALF = N // CORES
CH = 1000
NC = HALF // CH

_DN = (((1,), (1,)), ((), ()))


def _mlp3_kernel(x_hbm, w0_ref, w1_ref, w2_ref, b0_ref, b1_ref, b2_ref,
                 out_hbm, xbuf, obuf, insem, outsem):
    base = pl.program_id(0) * HALF

    def in_copy(i):
        return pltpu.make_async_copy(
            x_hbm.at[pl.ds(base + i * CH, CH), :], xbuf.at[i], insem.at[i])

    def out_copy(i):
        return pltpu.make_async_copy(
            obuf.at[i], out_hbm.at[pl.ds(base + i * CH, CH), :],
            outsem.at[i])

    for i in range(NC):
        in_copy(i).start()
    for i in range(NC):
        in_copy(i).wait()
        x = xbuf[i].astype(jnp.bfloat16)
        h = jax.lax.dot_general(x, w0_ref[...].astype(jnp.bfloat16), _DN,
                                preferred_element_type=jnp.float32)
        h = jnp.maximum(h + b0_ref[...], 0.0).astype(jnp.bfloat16)
        h = jax.lax.dot_general(h, w1_ref[...].astype(jnp.bfloat16), _DN,
                                preferred_element_type=jnp.float32)
        h = jnp.maximum(h + b1_ref[...], 0.0).astype(jnp.bfloat16)
        h = jax.lax.dot_general(h, w2_ref[...].astype(jnp.bfloat16), _DN,
                                preferred_element_type=jnp.float32)
        obuf[i] = h + b2_ref[...]
        out_copy(i).start()
    for i in range(NC):
        out_copy(i).wait()


def kernel(x, edge_index, edge_weight, W0, b0, W1, b1, W2, b2):
    full = pl.BlockSpec((D, D), lambda i: (0, 0))
    brow = pl.BlockSpec((1, D), lambda i: (0, 0))
    hbm = pl.BlockSpec(memory_space=pl.ANY)
    out = pl.pallas_call(
        _mlp3_kernel,
        grid=(CORES,),
        in_specs=[
            hbm,
            full, full, full,
            brow, brow, brow,
        ],
        out_specs=hbm,
        out_shape=jax.ShapeDtypeStruct((N, D), jnp.float32),
        scratch_shapes=[
            pltpu.VMEM((NC, CH, D), jnp.float32),
            pltpu.VMEM((NC, CH, D), jnp.float32),
            pltpu.SemaphoreType.DMA((NC,)),
            pltpu.SemaphoreType.DMA((NC,)),
        ],
        compiler_params=pltpu.CompilerParams(
            dimension_semantics=("parallel",),
        ),
    )(x, W0, W1, W2,
      b0.reshape(1, D), b1.reshape(1, D), b2.reshape(1, D))
    return out


# speedup vs baseline: 1.2271x; 1.2271x over previous
import jax
import jax.numpy as jnp
from jax.experimental import pallas as pl
from jax.experimental.pallas import tpu as pltpu

N = 10000
D = 128
CH = 2000
NC = N // CH

_DN = (((1,), (1,)), ((), ()))


def _mlp3_kernel(x_hbm, w0_ref, w1_ref, w2_ref, b0_ref, b1_ref, b2_ref,
                 out_hbm):
    w0 = w0_ref[...].astype(jnp.bfloat16)
    w1 = w1_ref[...].astype(jnp.bfloat16)
    w2 = w2_ref[...].astype(jnp.bfloat16)

    def chunk_body(x_ref, o_ref):
        x = x_ref[...].astype(jnp.bfloat16)
        h = jax.lax.dot_general(x, w0, _DN,
                                preferred_element_type=jnp.float32)
        h = jnp.maximum(h + b0_ref[...], 0.0).astype(jnp.bfloat16)
        h = jax.lax.dot_general(h, w1, _DN,
                                preferred_element_type=jnp.float32)
        h = jnp.maximum(h + b1_ref[...], 0.0).astype(jnp.bfloat16)
        h = jax.lax.dot_general(h, w2, _DN,
                                preferred_element_type=jnp.float32)
        o_ref[...] = h + b2_ref[...]

    pipeline = pltpu.emit_pipeline(
        chunk_body,
        grid=(NC,),
        in_specs=[pl.BlockSpec((CH, D), lambda i: (i, 0))],
        out_specs=[pl.BlockSpec((CH, D), lambda i: (i, 0))],
    )
    pipeline(x_hbm, out_hbm)


def kernel(x, edge_index, edge_weight, W0, b0, W1, b1, W2, b2):
    full = pl.BlockSpec((D, D), lambda: (0, 0))
    brow = pl.BlockSpec((1, D), lambda: (0, 0))
    hbm = pl.BlockSpec(memory_space=pl.ANY)
    out = pl.pallas_call(
        _mlp3_kernel,
        in_specs=[
            hbm,
            full, full, full,
            brow, brow, brow,
        ],
        out_specs=hbm,
        out_shape=jax.ShapeDtypeStruct((N, D), jnp.float32),
    )(x, W0, W1, W2,
      b0.reshape(1, D), b1.reshape(1, D), b2.reshape(1, D))
    return out


# speedup vs baseline: 1.3226x; 1.0778x over previous
import jax
import jax.numpy as jnp
from jax.experimental import pallas as pl
from jax.experimental.pallas import tpu as pltpu

N = 10000
D = 128

_DN = (((1,), (1,)), ((), ()))


def _mlp3_kernel(x_ref, w0_ref, w1_ref, w2_ref, b0_ref, b1_ref, b2_ref,
                 out_ref):
    x = x_ref[...].astype(jnp.bfloat16)
    h = jax.lax.dot_general(x, w0_ref[...].astype(jnp.bfloat16), _DN,
                            preferred_element_type=jnp.float32)
    h = jnp.maximum(h + b0_ref[...], 0.0).astype(jnp.bfloat16)
    h = jax.lax.dot_general(h, w1_ref[...].astype(jnp.bfloat16), _DN,
                            preferred_element_type=jnp.float32)
    h = jnp.maximum(h + b1_ref[...], 0.0).astype(jnp.bfloat16)
    h = jax.lax.dot_general(h, w2_ref[...].astype(jnp.bfloat16), _DN,
                            preferred_element_type=jnp.float32)
    out_ref[...] = h + b2_ref[...]


def kernel(x, edge_index, edge_weight, W0, b0, W1, b1, W2, b2):
    out = pl.pallas_call(
        _mlp3_kernel,
        out_shape=jax.ShapeDtypeStruct((N, D), jnp.float32),
    )(x, W0, W1, W2,
      b0.reshape(1, D), b1.reshape(1, D), b2.reshape(1, D))
    return out
